# Initial kernel scaffold; baseline (speedup 1.0000x reference)
import jax
import jax.numpy as jnp
from jax.experimental import pallas as pl


def kernel(xF, edge_index_F, batch_F, A_F, xD, edge_index_D, batch_D, A_D, TembF, SembF, TembD, SembD, WF1, bF1, WF2, bF2, WD1, bD1, WD2, bD2, lin1_W, lin1_b):
    raise NotImplementedError("write your pallas kernel here")


# speedup vs baseline: 2.8468x; 2.8468x over previous
import functools

import jax
import jax.numpy as jnp
from jax import lax
from jax.experimental import pallas as pl
from jax.experimental.pallas import tpu as pltpu
from jax.experimental.pallas import tpu_sc as plsc

NN = 36864
EE = 73728
DD = 1025
BB = 1024
RB = 288
GRID = NN // RB


def _embmm_body(e_ref, w_ref, b_ref, o_ref):
    acc = jnp.dot(e_ref[:].astype(jnp.bfloat16), w_ref[:],
                  preferred_element_type=jnp.float32)
    o_ref[:] = acc + b_ref[:]


def _embmm(emb8x, wcb, bpad, fo):
    return pl.pallas_call(
        _embmm_body,
        out_shape=jax.ShapeDtypeStruct((RB, 2 * fo), jnp.float32),
    )(emb8x, wcb, bpad)


def _mm1_body(x_ref, w_ref, e_ref, d_ref, u_ref, y_ref, *, fo):
    acc = jnp.dot(x_ref[:].astype(jnp.bfloat16), w_ref[:],
                  preferred_element_type=jnp.float32)
    acc = acc + e_ref[:]
    d = d_ref[:]
    dis = jnp.where(d > 0.0, lax.rsqrt(jnp.maximum(d, 1.0)), 0.0)
    u_ref[:] = acc[:, :fo]
    y_ref[:] = acc[:, fo:] * dis


def _mm1(x, wcb, emb8, deg2, fo):
    fi = x.shape[1]
    return pl.pallas_call(
        functools.partial(_mm1_body, fo=fo),
        grid=(GRID,),
        in_specs=[
            pl.BlockSpec((RB, fi), lambda i: (i, 0)),
            pl.BlockSpec((fi, 2 * fo), lambda i: (0, 0)),
            pl.BlockSpec((RB, 2 * fo), lambda i: (0, 0)),
            pl.BlockSpec((RB, 1), lambda i: (i, 0)),
        ],
        out_specs=[
            pl.BlockSpec((RB, fo), lambda i: (i, 0)),
            pl.BlockSpec((RB, fo), lambda i: (i, 0)),
        ],
        out_shape=[
            jax.ShapeDtypeStruct((NN, fo), jnp.float32),
            jax.ShapeDtypeStruct((NN, fo), jnp.float32),
        ],
    )(x, wcb, emb8, deg2)


def _stats_body(u_ref, s_ref, d_ref, h_ref, sm_ref, sq_ref):
    i = pl.program_id(0)

    @pl.when(i == 0)
    def _():
        sm_ref[:] = jnp.zeros_like(sm_ref)
        sq_ref[:] = jnp.zeros_like(sq_ref)

    d = d_ref[:]
    dis = jnp.where(d > 0.0, lax.rsqrt(jnp.maximum(d, 1.0)), 0.0)
    h = u_ref[:] - dis * s_ref[:]
    h_ref[:] = h
    sm_ref[:] += jnp.sum(h, axis=0, keepdims=True)
    sq_ref[:] += jnp.sum(h * h, axis=0, keepdims=True)


def _stats(u0, s, deg2, fo):
    return pl.pallas_call(
        _stats_body,
        grid=(GRID,),
        in_specs=[
            pl.BlockSpec((RB, fo), lambda i: (i, 0)),
            pl.BlockSpec((RB, fo), lambda i: (i, 0)),
            pl.BlockSpec((RB, 1), lambda i: (i, 0)),
        ],
        out_specs=[
            pl.BlockSpec((RB, fo), lambda i: (i, 0)),
            pl.BlockSpec((1, fo), lambda i: (0, 0)),
            pl.BlockSpec((1, fo), lambda i: (0, 0)),
        ],
        out_shape=[
            jax.ShapeDtypeStruct((NN, fo), jnp.float32),
            jax.ShapeDtypeStruct((1, fo), jnp.float32),
            jax.ShapeDtypeStruct((1, fo), jnp.float32),
        ],
    )(u0, s, deg2)


def _mm2_body(h_ref, sm_ref, sq_ref, w_ref, b_ref, d_ref, u_ref, y_ref, *, fo):
    m = sm_ref[:] * (1.0 / NN)
    var = sq_ref[:] * (1.0 / NN) - m * m
    inv = lax.rsqrt(var + 1e-5)
    h1 = jnp.maximum((h_ref[:] - m) * inv, 0.0)
    acc = jnp.dot(h1.astype(jnp.bfloat16), w_ref[:],
                  preferred_element_type=jnp.float32)
    acc = acc + b_ref[:]
    d = d_ref[:]
    dis = jnp.where(d > 0.0, lax.rsqrt(jnp.maximum(d, 1.0)), 0.0)
    u_ref[:] = acc[:, :fo]
    y_ref[:, :fo] = acc[:, fo:] * dis
    y_ref[:, fo:] = jnp.zeros_like(y_ref[:, fo:])


def _mm2(h, sm, sq, wcb, bpad, deg2, fo):
    fi = h.shape[1]
    return pl.pallas_call(
        functools.partial(_mm2_body, fo=fo),
        grid=(GRID,),
        in_specs=[
            pl.BlockSpec((RB, fi), lambda i: (i, 0)),
            pl.BlockSpec((1, fi), lambda i: (0, 0)),
            pl.BlockSpec((1, fi), lambda i: (0, 0)),
            pl.BlockSpec((fi, 2 * fo), lambda i: (0, 0)),
            pl.BlockSpec((1, 2 * fo), lambda i: (0, 0)),
            pl.BlockSpec((RB, 1), lambda i: (i, 0)),
        ],
        out_specs=[
            pl.BlockSpec((RB, fo), lambda i: (i, 0)),
            pl.BlockSpec((RB, 128), lambda i: (i, 0)),
        ],
        out_shape=[
            jax.ShapeDtypeStruct((NN, fo), jnp.float32),
            jax.ShapeDtypeStruct((NN, 128), jnp.float32),
        ],
    )(h, sm, sq, wcb, bpad, deg2)


def _stats2_body(u_ref, s_ref, d_ref, xs_ref, sm_ref, sq_ref):
    i = pl.program_id(0)

    @pl.when(i == 0)
    def _():
        sm_ref[:] = jnp.zeros_like(sm_ref)
        sq_ref[:] = jnp.zeros_like(sq_ref)

    d = d_ref[:]
    dis = jnp.where(d > 0.0, lax.rsqrt(jnp.maximum(d, 1.0)), 0.0)
    h = u_ref[:] - dis * s_ref[:]
    sm_ref[:] += jnp.sum(h, axis=0, keepdims=True)
    sq_ref[:] += jnp.sum(h * h, axis=0, keepdims=True)
    for g in range(RB // 36):
        xs_ref[g * 12:(g + 1) * 12, :] = h[g * 36 + 12:g * 36 + 24, :]


def _stats2(u0, s, deg2, fo):
    return pl.pallas_call(
        _stats2_body,
        grid=(GRID,),
        in_specs=[
            pl.BlockSpec((RB, fo), lambda i: (i, 0)),
            pl.BlockSpec((RB, fo), lambda i: (i, 0)),
            pl.BlockSpec((RB, 1), lambda i: (i, 0)),
        ],
        out_specs=[
            pl.BlockSpec((96, fo), lambda i: (i, 0)),
            pl.BlockSpec((1, fo), lambda i: (0, 0)),
            pl.BlockSpec((1, fo), lambda i: (0, 0)),
        ],
        out_shape=[
            jax.ShapeDtypeStruct((BB * 12, fo), jnp.float32),
            jax.ShapeDtypeStruct((1, fo), jnp.float32),
            jax.ShapeDtypeStruct((1, fo), jnp.float32),
        ],
    )(u0, s, deg2)


def _heads_body(xf_ref, smf_ref, sqf_ref, xd_ref, smd_ref, sqd_ref,
                lf_ref, ld_ref, lt_ref, lb_ref,
                of_ref, od_ref, fu_ref, oo_ref):
    def feat(x_ref, sm_ref, sq_ref):
        m = sm_ref[:] * (1.0 / NN)
        var = sq_ref[:] * (1.0 / NN) - m * m
        inv = lax.rsqrt(var + 1e-5)
        h = jnp.maximum((x_ref[:] - m[None, :, :]) * inv[None, :, :], 0.0)
        return jnp.concatenate(
            [jnp.max(h, axis=1), jnp.sum(h, axis=1) * (1.0 / 12.0)], axis=1)

    def smax(z):
        e = jnp.exp(z - jnp.max(z, axis=1, keepdims=True))
        return e / jnp.sum(e, axis=1, keepdims=True)

    featF = feat(xf_ref, smf_ref, sqf_ref)
    featD = feat(xd_ref, smd_ref, sqd_ref)
    lb = lb_ref[:]
    of_ref[:] = smax(jnp.dot(featF, lf_ref[:],
                             preferred_element_type=jnp.float32) + lb)
    od_ref[:] = smax(jnp.dot(featD, ld_ref[:],
                             preferred_element_type=jnp.float32) + lb)
    fu = jnp.concatenate([featF, featD], axis=1)
    fu_ref[:] = fu
    oo_ref[:] = smax(jnp.dot(fu, lt_ref[:],
                             preferred_element_type=jnp.float32) + lb)


def _heads(xsF3, smF, sqF, xsD3, smD, sqD, linFT, linDT, linT, lb2):
    gb = 128
    return pl.pallas_call(
        _heads_body,
        grid=(BB // gb,),
        in_specs=[
            pl.BlockSpec((gb, 12, 64), lambda i: (i, 0, 0)),
            pl.BlockSpec((1, 64), lambda i: (0, 0)),
            pl.BlockSpec((1, 64), lambda i: (0, 0)),
            pl.BlockSpec((gb, 12, 64), lambda i: (i, 0, 0)),
            pl.BlockSpec((1, 64), lambda i: (0, 0)),
            pl.BlockSpec((1, 64), lambda i: (0, 0)),
            pl.BlockSpec((128, 5), lambda i: (0, 0)),
            pl.BlockSpec((128, 5), lambda i: (0, 0)),
            pl.BlockSpec((256, 5), lambda i: (0, 0)),
            pl.BlockSpec((1, 5), lambda i: (0, 0)),
        ],
        out_specs=[
            pl.BlockSpec((gb, 5), lambda i: (i, 0)),
            pl.BlockSpec((gb, 5), lambda i: (i, 0)),
            pl.BlockSpec((gb, 256), lambda i: (i, 0)),
            pl.BlockSpec((gb, 5), lambda i: (i, 0)),
        ],
        out_shape=[
            jax.ShapeDtypeStruct((BB, 5), jnp.float32),
            jax.ShapeDtypeStruct((BB, 5), jnp.float32),
            jax.ShapeDtypeStruct((BB, 256), jnp.float32),
            jax.ShapeDtypeStruct((BB, 5), jnp.float32),
        ],
    )(xsF3, smF, sqF, xsD3, smD, sqD, linFT, linDT, linT, lb2)


EP = EE // 16
NSL = EP // 128


def _fill_const(ref, n, val, dtype):
    def body(i, _):
        ref[pl.ds(i * 16, 16)] = jnp.full((16,), val, dtype)
        return 0
    lax.fori_loop(0, n // 16, body, 0)


def _deg_kernel(rows):
    mesh = plsc.VectorSubcoreMesh(core_axis_name="c", subcore_axis_name="s")

    @functools.partial(
        pl.kernel, mesh=mesh,
        compiler_params=pltpu.CompilerParams(needs_layout_passes=False),
        out_type=jax.ShapeDtypeStruct((2, NN), jnp.float32),
        scratch_types=[
            pltpu.VMEM((NSL, 128), jnp.int32),
            pltpu.VMEM((128,), jnp.float32),
            pltpu.VMEM((NN // 16,), jnp.float32),
            pltpu.VMEM_SHARED((NN,), jnp.float32),
            pltpu.SemaphoreType.DMA,
        ],
    )
    def k(r_hbm, d_hbm, idx2, ones_v, zrow, deg_sh, sem):
        c = lax.axis_index("c")
        s = lax.axis_index("s")
        _fill_const(ones_v, 128, 1.0, jnp.float32)
        _fill_const(zrow, NN // 16, 0.0, jnp.float32)
        pltpu.sync_copy(r_hbm.at[c, s], idx2)
        pltpu.sync_copy(zrow, deg_sh.at[pl.ds(s * (NN // 16), NN // 16)])
        plsc.subcore_barrier()
        for j in range(NSL):
            pltpu.async_copy(ones_v, deg_sh.at[idx2.at[j]], sem, add=True)
        for j in range(NSL):
            pltpu.make_async_copy(ones_v, deg_sh.at[idx2.at[j]], sem).wait()
        plsc.subcore_barrier()
        pltpu.sync_copy(deg_sh.at[pl.ds(s * (NN // 16), NN // 16)],
                        d_hbm.at[c, pl.ds(s * (NN // 16), NN // 16)])

    return k(rows)


EPW = 4608
NW = EE // EPW
CAPL = 2048
GS = 64


def _scatter_kernel(fo, fg, rr, y, row, col):
    npass = (NN // 32) // rr
    mesh = plsc.VectorSubcoreMesh(core_axis_name="c", subcore_axis_name="s")

    @functools.partial(
        pl.kernel, mesh=mesh,
        compiler_params=pltpu.CompilerParams(needs_layout_passes=False),
        out_type=jax.ShapeDtypeStruct((NN * fo,), jnp.float32),
        scratch_types=[
            pltpu.VMEM((EPW,), jnp.int32),
            pltpu.VMEM((EPW,), jnp.int32),
            pltpu.VMEM((CAPL + 128,), jnp.int32),
            pltpu.VMEM((CAPL + 128,), jnp.int32),
            pltpu.VMEM((GS,), jnp.int32),
            pltpu.VMEM((GS, fg), jnp.float32),
            pltpu.VMEM(((rr + 32) * fo,), jnp.float32),
            pltpu.VMEM_SHARED((EE,), jnp.int32),
            pltpu.VMEM_SHARED((EE,), jnp.int32),
            pltpu.SemaphoreType.DMA,
        ],
    )
    def k(y_hbm, r_hbm, c_hbm, z_hbm, o_hbm,
          rowb, colb, crow1, cloc1, srow, gbuf, acc,
          rows_sh, cols_sh, sem):
        c = lax.axis_index("c")
        s = lax.axis_index("s")
        t = s * 2 + c
        lanes = lax.broadcasted_iota(jnp.int32, (16,), 0)
        pltpu.sync_copy(r_hbm.at[pl.ds(s * EP, EP)],
                        rows_sh.at[pl.ds(s * EP, EP)])
        pltpu.sync_copy(c_hbm.at[pl.ds(s * EP, EP)],
                        cols_sh.at[pl.ds(s * EP, EP)])
        plsc.subcore_barrier()

        def flush(cnt):
            for q in range(4):
                crow1[pl.ds(cnt + q * 16, 16)] = rowb[pl.ds(q * 16, 16)]
                cloc1[pl.ds(cnt + q * 16, 16)] = rr + (q % 2) * 16 + lanes
            nsl = (cnt + GS - 1) // GS

            def sl(g, _):
                for q in range(GS // 16):
                    srow[pl.ds(q * 16, 16)] = (
                        crow1[pl.ds(g * GS + q * 16, 16)])
                pltpu.async_copy(y_hbm.at[srow], gbuf, sem).wait()
                for q in range(GS // 16):
                    dv = cloc1[pl.ds(g * GS + q * 16, 16)] * fo

                    for l in range(16):
                        dst = dv[l]

                        def col_add(kq, _):
                            acc_slice = acc.at[pl.ds(dst + kq * 16, 16)]
                            plsc.addupdate(
                                acc_slice,
                                gbuf[q * 16 + l, pl.ds(kq * 16, 16)])
                            return 0

                        lax.fori_loop(0, fo // 16, col_add, 0)
                return 0

            lax.fori_loop(0, nsl, sl, 0)

        def one_pass(p, _):
            base = t * (NN // 32) + p * rr
            for q in range((rr + 32) // 32):
                pltpu.sync_copy(z_hbm, acc.at[pl.ds(q * 32 * fo, 32 * fo)])

            def window(w, cnt):
                pltpu.sync_copy(rows_sh.at[pl.ds(w * EPW, EPW)], rowb)
                pltpu.sync_copy(cols_sh.at[pl.ds(w * EPW, EPW)], colb)

                def comp(i, cnt):
                    cv = colb[pl.ds(i * 16, 16)]
                    rv = rowb[pl.ds(i * 16, 16)]
                    m = (cv >= base) & (cv < base + rr)
                    pos = cnt + plsc.cumsum(m.astype(jnp.int32)) - 1
                    plsc.store_scatter(crow1, [pos], rv, mask=m)
                    plsc.store_scatter(cloc1, [pos], cv - base, mask=m)
                    cnt = cnt + jnp.sum(m.astype(jnp.int32))

                    @pl.when(cnt >= CAPL)
                    def _():
                        flush(cnt)

                    return jnp.where(cnt >= CAPL, 0, cnt)

                return lax.fori_loop(0, EPW // 16, comp, cnt)

            cnt = lax.fori_loop(0, NW, window, 0)

            @pl.when(cnt > 0)
            def _():
                flush(cnt)

            pltpu.sync_copy(acc.at[pl.ds(0, rr * fo)],
                            o_hbm.at[pl.ds(base * fo, rr * fo)])
            return 0

        lax.fori_loop(0, npass, one_pass, 0)

    return k(y, row, col,
             jnp.zeros((32 * fo,), jnp.float32)).reshape(NN, fo)


def _emb_table(Temb, Semb):
    e = (Temb[:, None, :] + Semb[None, :, :]).reshape(36, DD)
    return jnp.tile(e, (RB // 36, 1))


def kernel(xF, edge_index_F, batch_F, A_F, xD, edge_index_D, batch_D, A_D,
           TembF, SembF, TembD, SembD, WF1, bF1, WF2, bF2, WD1, bD1,
           WD2, bD2, lin1_W, lin1_b):
    f32 = jnp.float32
    rowF = edge_index_F[0]
    colF = edge_index_F[1]
    rowD = edge_index_D[0]
    colD = edge_index_D[1]

    rows = jnp.stack([rowF, rowD]).reshape(2, 16, EE // (16 * 128), 128)
    deg = _deg_kernel(rows)
    degF2 = deg[0][:, None]
    degD2 = deg[1][:, None]

    wF1 = jnp.concatenate([WF1[0], WF1[1]], axis=1).astype(jnp.bfloat16)
    wD1 = jnp.concatenate([WD1[0], WD1[1]], axis=1).astype(jnp.bfloat16)
    bF1p = jnp.concatenate([bF1, jnp.zeros((256,), f32)])[None, :]
    bD1p = jnp.concatenate([bD1, jnp.zeros((256,), f32)])[None, :]
    embF8 = _embmm(_emb_table(TembF, SembF), wF1, bF1p, 256)
    embD8 = _embmm(_emb_table(TembD, SembD), wD1, bD1p, 256)

    u0F, ypF = _mm1(xF, wF1, embF8, degF2, 256)
    u0D, ypD = _mm1(xD, wD1, embD8, degD2, 256)

    sF = _scatter_kernel(256, 256, 288, ypF, rowF, colF)
    sD = _scatter_kernel(256, 256, 288, ypD, rowD, colD)

    hF, smF, sqF = _stats(u0F, sF, degF2, 256)
    hD, smD, sqD = _stats(u0D, sD, degD2, 256)

    wF2 = jnp.concatenate([WF2[0], WF2[1]], axis=1).astype(jnp.bfloat16)
    wD2 = jnp.concatenate([WD2[0], WD2[1]], axis=1).astype(jnp.bfloat16)
    bF2p = jnp.concatenate([bF2, jnp.zeros((64,), f32)])[None, :]
    bD2p = jnp.concatenate([bD2, jnp.zeros((64,), f32)])[None, :]

    u0F2, ypF2 = _mm2(hF, smF, sqF, wF2, bF2p, degF2, 64)
    u0D2, ypD2 = _mm2(hD, smD, sqD, wD2, bD2p, degD2, 64)

    sF2 = _scatter_kernel(64, 128, 1152, ypF2, rowF, colF)
    sD2 = _scatter_kernel(64, 128, 1152, ypD2, rowD, colD)

    xsF, smF2, sqF2 = _stats2(u0F2, sF2, degF2, 64)
    xsD, smD2, sqD2 = _stats2(u0D2, sD2, degD2, 64)

    linFT = lin1_W[:, :128].T
    linDT = lin1_W[:, 128:].T
    linT = lin1_W.T
    lb2 = lin1_b[None, :]

    outputF, outputD, fusion, output = _heads(
        xsF.reshape(BB, 12, 64), smF2, sqF2,
        xsD.reshape(BB, 12, 64), smD2, sqD2,
        linFT, linDT, linT, lb2)
    return (outputF, outputD, fusion, output)


# speedup vs baseline: 3.5387x; 1.2430x over previous
import functools

import jax
import jax.numpy as jnp
from jax import lax
from jax.experimental import pallas as pl
from jax.experimental.pallas import tpu as pltpu
from jax.experimental.pallas import tpu_sc as plsc

NN = 36864
EE = 73728
DD = 1025
BB = 1024
RB = 288
GRID = NN // RB


def _embmm_body(e_ref, w_ref, b_ref, o_ref):
    acc = jnp.dot(e_ref[:].astype(jnp.bfloat16), w_ref[:],
                  preferred_element_type=jnp.float32)
    o_ref[:] = acc + b_ref[:]


def _embmm(emb8x, wcb, bpad, fo):
    return pl.pallas_call(
        _embmm_body,
        out_shape=jax.ShapeDtypeStruct((RB, 2 * fo), jnp.float32),
    )(emb8x, wcb, bpad)


def _mm1_body(x_ref, w_ref, e_ref, d_ref, u_ref, y_ref, *, fo):
    acc = jnp.dot(x_ref[:].astype(jnp.bfloat16), w_ref[:],
                  preferred_element_type=jnp.float32)
    acc = acc + e_ref[:]
    d = d_ref[:]
    dis = jnp.where(d > 0.0, lax.rsqrt(jnp.maximum(d, 1.0)), 0.0)
    u_ref[:] = acc[:, :fo]
    y_ref[:] = acc[:, fo:] * dis


def _mm1(x, wcb, emb8, deg2, fo):
    fi = x.shape[1]
    return pl.pallas_call(
        functools.partial(_mm1_body, fo=fo),
        grid=(GRID,),
        in_specs=[
            pl.BlockSpec((RB, fi), lambda i: (i, 0)),
            pl.BlockSpec((fi, 2 * fo), lambda i: (0, 0)),
            pl.BlockSpec((RB, 2 * fo), lambda i: (0, 0)),
            pl.BlockSpec((RB, 1), lambda i: (i, 0)),
        ],
        out_specs=[
            pl.BlockSpec((RB, fo), lambda i: (i, 0)),
            pl.BlockSpec((RB, fo), lambda i: (i, 0)),
        ],
        out_shape=[
            jax.ShapeDtypeStruct((NN, fo), jnp.float32),
            jax.ShapeDtypeStruct((NN, fo), jnp.float32),
        ],
    )(x, wcb, emb8, deg2)


def _stats_body(u_ref, s_ref, d_ref, h_ref, sm_ref, sq_ref):
    i = pl.program_id(0)

    @pl.when(i == 0)
    def _():
        sm_ref[:] = jnp.zeros_like(sm_ref)
        sq_ref[:] = jnp.zeros_like(sq_ref)

    d = d_ref[:]
    dis = jnp.where(d > 0.0, lax.rsqrt(jnp.maximum(d, 1.0)), 0.0)
    h = u_ref[:] - dis * s_ref[:]
    h_ref[:] = h
    sm_ref[:] += jnp.sum(h, axis=0, keepdims=True)
    sq_ref[:] += jnp.sum(h * h, axis=0, keepdims=True)


def _stats(u0, s, deg2, fo):
    return pl.pallas_call(
        _stats_body,
        grid=(GRID,),
        in_specs=[
            pl.BlockSpec((RB, fo), lambda i: (i, 0)),
            pl.BlockSpec((RB, fo), lambda i: (i, 0)),
            pl.BlockSpec((RB, 1), lambda i: (i, 0)),
        ],
        out_specs=[
            pl.BlockSpec((RB, fo), lambda i: (i, 0)),
            pl.BlockSpec((1, fo), lambda i: (0, 0)),
            pl.BlockSpec((1, fo), lambda i: (0, 0)),
        ],
        out_shape=[
            jax.ShapeDtypeStruct((NN, fo), jnp.float32),
            jax.ShapeDtypeStruct((1, fo), jnp.float32),
            jax.ShapeDtypeStruct((1, fo), jnp.float32),
        ],
    )(u0, s, deg2)


def _mm2_body(h_ref, sm_ref, sq_ref, w_ref, b_ref, d_ref, u_ref, y_ref, *, fo):
    m = sm_ref[:] * (1.0 / NN)
    var = sq_ref[:] * (1.0 / NN) - m * m
    inv = lax.rsqrt(var + 1e-5)
    h1 = jnp.maximum((h_ref[:] - m) * inv, 0.0)
    acc = jnp.dot(h1.astype(jnp.bfloat16), w_ref[:],
                  preferred_element_type=jnp.float32)
    acc = acc + b_ref[:]
    d = d_ref[:]
    dis = jnp.where(d > 0.0, lax.rsqrt(jnp.maximum(d, 1.0)), 0.0)
    u_ref[:] = acc[:, :fo]
    y_ref[:, :fo] = acc[:, fo:] * dis
    y_ref[:, fo:] = jnp.zeros_like(y_ref[:, fo:])


def _mm2(h, sm, sq, wcb, bpad, deg2, fo):
    fi = h.shape[1]
    return pl.pallas_call(
        functools.partial(_mm2_body, fo=fo),
        grid=(GRID,),
        in_specs=[
            pl.BlockSpec((RB, fi), lambda i: (i, 0)),
            pl.BlockSpec((1, fi), lambda i: (0, 0)),
            pl.BlockSpec((1, fi), lambda i: (0, 0)),
            pl.BlockSpec((fi, 2 * fo), lambda i: (0, 0)),
            pl.BlockSpec((1, 2 * fo), lambda i: (0, 0)),
            pl.BlockSpec((RB, 1), lambda i: (i, 0)),
        ],
        out_specs=[
            pl.BlockSpec((RB, fo), lambda i: (i, 0)),
            pl.BlockSpec((RB, 128), lambda i: (i, 0)),
        ],
        out_shape=[
            jax.ShapeDtypeStruct((NN, fo), jnp.float32),
            jax.ShapeDtypeStruct((NN, 128), jnp.float32),
        ],
    )(h, sm, sq, wcb, bpad, deg2)


def _stats2_body(u_ref, s_ref, d_ref, xs_ref, sm_ref, sq_ref):
    i = pl.program_id(0)

    @pl.when(i == 0)
    def _():
        sm_ref[:] = jnp.zeros_like(sm_ref)
        sq_ref[:] = jnp.zeros_like(sq_ref)

    d = d_ref[:]
    dis = jnp.where(d > 0.0, lax.rsqrt(jnp.maximum(d, 1.0)), 0.0)
    h = u_ref[:] - dis * s_ref[:]
    sm_ref[:] += jnp.sum(h, axis=0, keepdims=True)
    sq_ref[:] += jnp.sum(h * h, axis=0, keepdims=True)
    for g in range(RB // 36):
        xs_ref[g * 12:(g + 1) * 12, :] = h[g * 36 + 12:g * 36 + 24, :]


def _stats2(u0, s, deg2, fo):
    return pl.pallas_call(
        _stats2_body,
        grid=(GRID,),
        in_specs=[
            pl.BlockSpec((RB, fo), lambda i: (i, 0)),
            pl.BlockSpec((RB, fo), lambda i: (i, 0)),
            pl.BlockSpec((RB, 1), lambda i: (i, 0)),
        ],
        out_specs=[
            pl.BlockSpec((96, fo), lambda i: (i, 0)),
            pl.BlockSpec((1, fo), lambda i: (0, 0)),
            pl.BlockSpec((1, fo), lambda i: (0, 0)),
        ],
        out_shape=[
            jax.ShapeDtypeStruct((BB * 12, fo), jnp.float32),
            jax.ShapeDtypeStruct((1, fo), jnp.float32),
            jax.ShapeDtypeStruct((1, fo), jnp.float32),
        ],
    )(u0, s, deg2)


def _heads_body(xf_ref, smf_ref, sqf_ref, xd_ref, smd_ref, sqd_ref,
                lf_ref, ld_ref, lt_ref, lb_ref,
                of_ref, od_ref, fu_ref, oo_ref):
    def feat(x_ref, sm_ref, sq_ref):
        m = sm_ref[:] * (1.0 / NN)
        var = sq_ref[:] * (1.0 / NN) - m * m
        inv = lax.rsqrt(var + 1e-5)
        h = jnp.maximum((x_ref[:] - m[None, :, :]) * inv[None, :, :], 0.0)
        return jnp.concatenate(
            [jnp.max(h, axis=1), jnp.sum(h, axis=1) * (1.0 / 12.0)], axis=1)

    def smax(z):
        e = jnp.exp(z - jnp.max(z, axis=1, keepdims=True))
        return e / jnp.sum(e, axis=1, keepdims=True)

    featF = feat(xf_ref, smf_ref, sqf_ref)
    featD = feat(xd_ref, smd_ref, sqd_ref)
    lb = lb_ref[:]
    of_ref[:] = smax(jnp.dot(featF, lf_ref[:],
                             preferred_element_type=jnp.float32) + lb)
    od_ref[:] = smax(jnp.dot(featD, ld_ref[:],
                             preferred_element_type=jnp.float32) + lb)
    fu = jnp.concatenate([featF, featD], axis=1)
    fu_ref[:] = fu
    oo_ref[:] = smax(jnp.dot(fu, lt_ref[:],
                             preferred_element_type=jnp.float32) + lb)


def _heads(xsF3, smF, sqF, xsD3, smD, sqD, linFT, linDT, linT, lb2):
    gb = 128
    return pl.pallas_call(
        _heads_body,
        grid=(BB // gb,),
        in_specs=[
            pl.BlockSpec((gb, 12, 64), lambda i: (i, 0, 0)),
            pl.BlockSpec((1, 64), lambda i: (0, 0)),
            pl.BlockSpec((1, 64), lambda i: (0, 0)),
            pl.BlockSpec((gb, 12, 64), lambda i: (i, 0, 0)),
            pl.BlockSpec((1, 64), lambda i: (0, 0)),
            pl.BlockSpec((1, 64), lambda i: (0, 0)),
            pl.BlockSpec((128, 5), lambda i: (0, 0)),
            pl.BlockSpec((128, 5), lambda i: (0, 0)),
            pl.BlockSpec((256, 5), lambda i: (0, 0)),
            pl.BlockSpec((1, 5), lambda i: (0, 0)),
        ],
        out_specs=[
            pl.BlockSpec((gb, 5), lambda i: (i, 0)),
            pl.BlockSpec((gb, 5), lambda i: (i, 0)),
            pl.BlockSpec((gb, 256), lambda i: (i, 0)),
            pl.BlockSpec((gb, 5), lambda i: (i, 0)),
        ],
        out_shape=[
            jax.ShapeDtypeStruct((BB, 5), jnp.float32),
            jax.ShapeDtypeStruct((BB, 5), jnp.float32),
            jax.ShapeDtypeStruct((BB, 256), jnp.float32),
            jax.ShapeDtypeStruct((BB, 5), jnp.float32),
        ],
    )(xsF3, smF, sqF, xsD3, smD, sqD, linFT, linDT, linT, lb2)


EP = EE // 16
NSL = EP // 128


def _fill_const(ref, n, val, dtype):
    def body(i, _):
        ref[pl.ds(i * 16, 16)] = jnp.full((16,), val, dtype)
        return 0
    lax.fori_loop(0, n // 16, body, 0)


def _deg_kernel(rows):
    mesh = plsc.VectorSubcoreMesh(core_axis_name="c", subcore_axis_name="s")

    @functools.partial(
        pl.kernel, mesh=mesh,
        compiler_params=pltpu.CompilerParams(needs_layout_passes=False),
        out_type=jax.ShapeDtypeStruct((2, NN), jnp.float32),
        scratch_types=[
            pltpu.VMEM((NSL, 128), jnp.int32),
            pltpu.VMEM((128,), jnp.float32),
            pltpu.VMEM((NN // 16,), jnp.float32),
            pltpu.VMEM_SHARED((NN,), jnp.float32),
            pltpu.SemaphoreType.DMA,
        ],
    )
    def k(r_hbm, d_hbm, idx2, ones_v, zrow, deg_sh, sem):
        c = lax.axis_index("c")
        s = lax.axis_index("s")
        _fill_const(ones_v, 128, 1.0, jnp.float32)
        _fill_const(zrow, NN // 16, 0.0, jnp.float32)
        pltpu.sync_copy(r_hbm.at[c, s], idx2)
        pltpu.sync_copy(zrow, deg_sh.at[pl.ds(s * (NN // 16), NN // 16)])
        plsc.subcore_barrier()
        for j in range(NSL):
            pltpu.async_copy(ones_v, deg_sh.at[idx2.at[j]], sem, add=True)
        for j in range(NSL):
            pltpu.make_async_copy(ones_v, deg_sh.at[idx2.at[j]], sem).wait()
        plsc.subcore_barrier()
        pltpu.sync_copy(deg_sh.at[pl.ds(s * (NN // 16), NN // 16)],
                        d_hbm.at[c, pl.ds(s * (NN // 16), NN // 16)])

    return k(rows)


EPW = 4608
NW = EE // EPW
CAPL = 8192
GS = 32


def _scatter_kernel(fo, fg, rr, y, row, col):
    npass = (NN // 32) // rr
    mesh = plsc.VectorSubcoreMesh(core_axis_name="c", subcore_axis_name="s")

    @functools.partial(
        pl.kernel, mesh=mesh,
        compiler_params=pltpu.CompilerParams(needs_layout_passes=False),
        out_type=jax.ShapeDtypeStruct((NN * fo,), jnp.float32),
        scratch_types=[
            pltpu.VMEM((EPW,), jnp.int32),
            pltpu.VMEM((EPW,), jnp.int32),
            pltpu.VMEM((CAPL + 128,), jnp.int32),
            pltpu.VMEM((CAPL + 128,), jnp.int32),
            pltpu.VMEM((GS,), jnp.int32),
            pltpu.VMEM((GS,), jnp.int32),
            pltpu.VMEM((GS, fg), jnp.float32),
            pltpu.VMEM((GS, fg), jnp.float32),
            pltpu.VMEM(((rr + 32) * fo,), jnp.float32),
            pltpu.SemaphoreType.DMA,
            pltpu.SemaphoreType.DMA,
        ],
    )
    def k(y_hbm, r_hbm, c_hbm, o_hbm,
          rowb, colb, crow1, cloc1, srowA, srowB, gbufA, gbufB, acc,
          semA, semB):
        c = lax.axis_index("c")
        s = lax.axis_index("s")
        t = s * 2 + c
        lanes = lax.broadcasted_iota(jnp.int32, (16,), 0)

        def load_idx(g, sr):
            for q in range(GS // 16):
                sr[pl.ds(q * 16, 16)] = crow1[pl.ds(g * GS + q * 16, 16)]

        def addg(g, gbuf):
            for q in range(GS // 16):
                dv = cloc1[pl.ds(g * GS + q * 16, 16)] * fo

                for l in range(16):
                    dst = dv[l]

                    def col_add(kq, _):
                        plsc.addupdate(
                            acc.at[pl.ds(dst + kq * 16, 16)],
                            gbuf[q * 16 + l, pl.ds(kq * 16, 16)])
                        return 0

                    lax.fori_loop(0, fo // 16, col_add, 0)

        def flush(cnt):
            for q in range(4):
                crow1[pl.ds(cnt + q * 16, 16)] = rowb[pl.ds(q * 16, 16)]
                cloc1[pl.ds(cnt + q * 16, 16)] = rr + (q % 2) * 16 + lanes
            nsl = (cnt + GS - 1) // GS
            load_idx(0, srowA)
            pltpu.async_copy(y_hbm.at[srowA], gbufA, semA)

            def sl(g, _):
                @pl.when(g % 2 == 0)
                def _():
                    pltpu.make_async_copy(y_hbm.at[srowA], gbufA, semA).wait()

                    @pl.when(g + 1 < nsl)
                    def _():
                        load_idx(g + 1, srowB)
                        pltpu.async_copy(y_hbm.at[srowB], gbufB, semB)
                    addg(g, gbufA)

                @pl.when(g % 2 == 1)
                def _():
                    pltpu.make_async_copy(y_hbm.at[srowB], gbufB, semB).wait()

                    @pl.when(g + 1 < nsl)
                    def _():
                        load_idx(g + 1, srowA)
                        pltpu.async_copy(y_hbm.at[srowA], gbufA, semA)
                    addg(g, gbufB)

                return 0

            lax.fori_loop(0, nsl, sl, 0)

        def one_pass(p, _):
            base = t * (NN // 32) + p * rr
            _fill_const(acc, (rr + 32) * fo, 0.0, jnp.float32)

            def window(w, cnt):
                pltpu.sync_copy(r_hbm.at[pl.ds(w * EPW, EPW)], rowb)
                pltpu.sync_copy(c_hbm.at[pl.ds(w * EPW, EPW)], colb)

                def comp(i, cnt):
                    cv = colb[pl.ds(i * 16, 16)]
                    mv = cv - base
                    m = (mv >= 0) & (mv < rr)
                    pc = jnp.sum(m.astype(jnp.int32))

                    @pl.when(pc > 0)
                    def _():
                        rv = rowb[pl.ds(i * 16, 16)]
                        pos = cnt + plsc.cumsum(m.astype(jnp.int32)) - 1
                        plsc.store_scatter(crow1, [pos], rv, mask=m)
                        plsc.store_scatter(cloc1, [pos], mv, mask=m)

                    return cnt + pc

                cnt = lax.fori_loop(0, EPW // 16, comp, cnt, unroll=2)

                @pl.when(cnt >= CAPL - EPW - 64)
                def _():
                    flush(cnt)

                return jnp.where(cnt >= CAPL - EPW - 64, 0, cnt)

            cnt = lax.fori_loop(0, NW, window, 0)

            @pl.when(cnt > 0)
            def _():
                flush(cnt)

            pltpu.sync_copy(acc.at[pl.ds(0, rr * fo)],
                            o_hbm.at[pl.ds(base * fo, rr * fo)])
            return 0

        lax.fori_loop(0, npass, one_pass, 0)

    return k(y, row, col).reshape(NN, fo)


def _emb_table(Temb, Semb):
    e = (Temb[:, None, :] + Semb[None, :, :]).reshape(36, DD)
    return jnp.tile(e, (RB // 36, 1))


def kernel(xF, edge_index_F, batch_F, A_F, xD, edge_index_D, batch_D, A_D,
           TembF, SembF, TembD, SembD, WF1, bF1, WF2, bF2, WD1, bD1,
           WD2, bD2, lin1_W, lin1_b):
    f32 = jnp.float32
    rowF = edge_index_F[0]
    colF = edge_index_F[1]
    rowD = edge_index_D[0]
    colD = edge_index_D[1]

    rows = jnp.stack([rowF, rowD]).reshape(2, 16, EE // (16 * 128), 128)
    deg = _deg_kernel(rows)
    degF2 = deg[0][:, None]
    degD2 = deg[1][:, None]

    wF1 = jnp.concatenate([WF1[0], WF1[1]], axis=1).astype(jnp.bfloat16)
    wD1 = jnp.concatenate([WD1[0], WD1[1]], axis=1).astype(jnp.bfloat16)
    bF1p = jnp.concatenate([bF1, jnp.zeros((256,), f32)])[None, :]
    bD1p = jnp.concatenate([bD1, jnp.zeros((256,), f32)])[None, :]
    embF8 = _embmm(_emb_table(TembF, SembF), wF1, bF1p, 256)
    embD8 = _embmm(_emb_table(TembD, SembD), wD1, bD1p, 256)

    u0F, ypF = _mm1(xF, wF1, embF8, degF2, 256)
    u0D, ypD = _mm1(xD, wD1, embD8, degD2, 256)

    sF = _scatter_kernel(256, 256, 288, ypF, rowF, colF)
    sD = _scatter_kernel(256, 256, 288, ypD, rowD, colD)

    hF, smF, sqF = _stats(u0F, sF, degF2, 256)
    hD, smD, sqD = _stats(u0D, sD, degD2, 256)

    wF2 = jnp.concatenate([WF2[0], WF2[1]], axis=1).astype(jnp.bfloat16)
    wD2 = jnp.concatenate([WD2[0], WD2[1]], axis=1).astype(jnp.bfloat16)
    bF2p = jnp.concatenate([bF2, jnp.zeros((64,), f32)])[None, :]
    bD2p = jnp.concatenate([bD2, jnp.zeros((64,), f32)])[None, :]

    u0F2, ypF2 = _mm2(hF, smF, sqF, wF2, bF2p, degF2, 64)
    u0D2, ypD2 = _mm2(hD, smD, sqD, wD2, bD2p, degD2, 64)

    sF2 = _scatter_kernel(64, 128, 1152, ypF2, rowF, colF)
    sD2 = _scatter_kernel(64, 128, 1152, ypD2, rowD, colD)

    xsF, smF2, sqF2 = _stats2(u0F2, sF2, degF2, 64)
    xsD, smD2, sqD2 = _stats2(u0D2, sD2, degD2, 64)

    linFT = lin1_W[:, :128].T
    linDT = lin1_W[:, 128:].T
    linT = lin1_W.T
    lb2 = lin1_b[None, :]

    outputF, outputD, fusion, output = _heads(
        xsF.reshape(BB, 12, 64), smF2, sqF2,
        xsD.reshape(BB, 12, 64), smD2, sqD2,
        linFT, linDT, linT, lb2)
    return (outputF, outputD, fusion, output)


# speedup vs baseline: 4.3055x; 1.2167x over previous
import functools

import jax
import jax.numpy as jnp
from jax import lax
from jax.experimental import pallas as pl
from jax.experimental.pallas import tpu as pltpu
from jax.experimental.pallas import tpu_sc as plsc

NN = 36864
EE = 73728
DD = 1025
BB = 1024
RB = 288
GRID = NN // RB


def _embmm_body(e_ref, w_ref, b_ref, o_ref):
    acc = jnp.dot(e_ref[:].astype(jnp.bfloat16), w_ref[:],
                  preferred_element_type=jnp.float32)
    o_ref[:] = acc + b_ref[:]


def _embmm(emb8x, wcb, bpad, fo):
    return pl.pallas_call(
        _embmm_body,
        out_shape=jax.ShapeDtypeStruct((RB, 2 * fo), jnp.float32),
    )(emb8x, wcb, bpad)


def _mm1_body(x_ref, w_ref, e_ref, d_ref, u_ref, y_ref, *, fo):
    acc = jnp.dot(x_ref[:].astype(jnp.bfloat16), w_ref[:],
                  preferred_element_type=jnp.float32)
    acc = acc + e_ref[:]
    d = d_ref[:]
    dis = jnp.where(d > 0.0, lax.rsqrt(jnp.maximum(d, 1.0)), 0.0)
    u_ref[:] = acc[:, :fo]
    y_ref[:] = acc[:, fo:] * dis


def _mm1(x, wcb, emb8, deg2, fo):
    fi = x.shape[1]
    return pl.pallas_call(
        functools.partial(_mm1_body, fo=fo),
        grid=(GRID,),
        in_specs=[
            pl.BlockSpec((RB, fi), lambda i: (i, 0)),
            pl.BlockSpec((fi, 2 * fo), lambda i: (0, 0)),
            pl.BlockSpec((RB, 2 * fo), lambda i: (0, 0)),
            pl.BlockSpec((RB, 1), lambda i: (i, 0)),
        ],
        out_specs=[
            pl.BlockSpec((RB, fo), lambda i: (i, 0)),
            pl.BlockSpec((RB, fo), lambda i: (i, 0)),
        ],
        out_shape=[
            jax.ShapeDtypeStruct((NN, fo), jnp.float32),
            jax.ShapeDtypeStruct((NN, fo), jnp.float32),
        ],
    )(x, wcb, emb8, deg2)


def _stats_body(u_ref, s_ref, d_ref, h_ref, sm_ref, sq_ref):
    i = pl.program_id(0)

    @pl.when(i == 0)
    def _():
        sm_ref[:] = jnp.zeros_like(sm_ref)
        sq_ref[:] = jnp.zeros_like(sq_ref)

    d = d_ref[:]
    dis = jnp.where(d > 0.0, lax.rsqrt(jnp.maximum(d, 1.0)), 0.0)
    h = u_ref[:] - dis * s_ref[:]
    h_ref[:] = h
    sm_ref[:] += jnp.sum(h, axis=0, keepdims=True)
    sq_ref[:] += jnp.sum(h * h, axis=0, keepdims=True)


def _stats(u0, s, deg2, fo):
    return pl.pallas_call(
        _stats_body,
        grid=(GRID,),
        in_specs=[
            pl.BlockSpec((RB, fo), lambda i: (i, 0)),
            pl.BlockSpec((RB, fo), lambda i: (i, 0)),
            pl.BlockSpec((RB, 1), lambda i: (i, 0)),
        ],
        out_specs=[
            pl.BlockSpec((RB, fo), lambda i: (i, 0)),
            pl.BlockSpec((1, fo), lambda i: (0, 0)),
            pl.BlockSpec((1, fo), lambda i: (0, 0)),
        ],
        out_shape=[
            jax.ShapeDtypeStruct((NN, fo), jnp.float32),
            jax.ShapeDtypeStruct((1, fo), jnp.float32),
            jax.ShapeDtypeStruct((1, fo), jnp.float32),
        ],
    )(u0, s, deg2)


def _mm2_body(h_ref, sm_ref, sq_ref, w_ref, b_ref, d_ref, u_ref, y_ref, *, fo):
    m = sm_ref[:] * (1.0 / NN)
    var = sq_ref[:] * (1.0 / NN) - m * m
    inv = lax.rsqrt(var + 1e-5)
    h1 = jnp.maximum((h_ref[:] - m) * inv, 0.0)
    acc = jnp.dot(h1.astype(jnp.bfloat16), w_ref[:],
                  preferred_element_type=jnp.float32)
    acc = acc + b_ref[:]
    d = d_ref[:]
    dis = jnp.where(d > 0.0, lax.rsqrt(jnp.maximum(d, 1.0)), 0.0)
    u_ref[:] = acc[:, :fo]
    y_ref[:, :fo] = acc[:, fo:] * dis
    y_ref[:, fo:] = jnp.zeros_like(y_ref[:, fo:])


def _mm2(h, sm, sq, wcb, bpad, deg2, fo):
    fi = h.shape[1]
    return pl.pallas_call(
        functools.partial(_mm2_body, fo=fo),
        grid=(GRID,),
        in_specs=[
            pl.BlockSpec((RB, fi), lambda i: (i, 0)),
            pl.BlockSpec((1, fi), lambda i: (0, 0)),
            pl.BlockSpec((1, fi), lambda i: (0, 0)),
            pl.BlockSpec((fi, 2 * fo), lambda i: (0, 0)),
            pl.BlockSpec((1, 2 * fo), lambda i: (0, 0)),
            pl.BlockSpec((RB, 1), lambda i: (i, 0)),
        ],
        out_specs=[
            pl.BlockSpec((RB, fo), lambda i: (i, 0)),
            pl.BlockSpec((RB, 128), lambda i: (i, 0)),
        ],
        out_shape=[
            jax.ShapeDtypeStruct((NN, fo), jnp.float32),
            jax.ShapeDtypeStruct((NN, 128), jnp.float32),
        ],
    )(h, sm, sq, wcb, bpad, deg2)


def _stats2_body(u_ref, s_ref, d_ref, xs_ref, sm_ref, sq_ref):
    i = pl.program_id(0)

    @pl.when(i == 0)
    def _():
        sm_ref[:] = jnp.zeros_like(sm_ref)
        sq_ref[:] = jnp.zeros_like(sq_ref)

    d = d_ref[:]
    dis = jnp.where(d > 0.0, lax.rsqrt(jnp.maximum(d, 1.0)), 0.0)
    h = u_ref[:] - dis * s_ref[:]
    sm_ref[:] += jnp.sum(h, axis=0, keepdims=True)
    sq_ref[:] += jnp.sum(h * h, axis=0, keepdims=True)
    for g in range(RB // 36):
        xs_ref[g * 12:(g + 1) * 12, :] = h[g * 36 + 12:g * 36 + 24, :]


def _stats2(u0, s, deg2, fo):
    return pl.pallas_call(
        _stats2_body,
        grid=(GRID,),
        in_specs=[
            pl.BlockSpec((RB, fo), lambda i: (i, 0)),
            pl.BlockSpec((RB, fo), lambda i: (i, 0)),
            pl.BlockSpec((RB, 1), lambda i: (i, 0)),
        ],
        out_specs=[
            pl.BlockSpec((96, fo), lambda i: (i, 0)),
            pl.BlockSpec((1, fo), lambda i: (0, 0)),
            pl.BlockSpec((1, fo), lambda i: (0, 0)),
        ],
        out_shape=[
            jax.ShapeDtypeStruct((BB * 12, fo), jnp.float32),
            jax.ShapeDtypeStruct((1, fo), jnp.float32),
            jax.ShapeDtypeStruct((1, fo), jnp.float32),
        ],
    )(u0, s, deg2)


def _heads_body(xf_ref, smf_ref, sqf_ref, xd_ref, smd_ref, sqd_ref,
                lf_ref, ld_ref, lt_ref, lb_ref,
                of_ref, od_ref, fu_ref, oo_ref):
    def feat(x_ref, sm_ref, sq_ref):
        m = sm_ref[:] * (1.0 / NN)
        var = sq_ref[:] * (1.0 / NN) - m * m
        inv = lax.rsqrt(var + 1e-5)
        h = jnp.maximum((x_ref[:] - m[None, :, :]) * inv[None, :, :], 0.0)
        return jnp.concatenate(
            [jnp.max(h, axis=1), jnp.sum(h, axis=1) * (1.0 / 12.0)], axis=1)

    def smax(z):
        e = jnp.exp(z - jnp.max(z, axis=1, keepdims=True))
        return e / jnp.sum(e, axis=1, keepdims=True)

    featF = feat(xf_ref, smf_ref, sqf_ref)
    featD = feat(xd_ref, smd_ref, sqd_ref)
    lb = lb_ref[:]
    of_ref[:] = smax(jnp.dot(featF, lf_ref[:],
                             preferred_element_type=jnp.float32) + lb)
    od_ref[:] = smax(jnp.dot(featD, ld_ref[:],
                             preferred_element_type=jnp.float32) + lb)
    fu = jnp.concatenate([featF, featD], axis=1)
    fu_ref[:] = fu
    oo_ref[:] = smax(jnp.dot(fu, lt_ref[:],
                             preferred_element_type=jnp.float32) + lb)


def _heads(xsF3, smF, sqF, xsD3, smD, sqD, linFT, linDT, linT, lb2):
    gb = 128
    return pl.pallas_call(
        _heads_body,
        grid=(BB // gb,),
        in_specs=[
            pl.BlockSpec((gb, 12, 64), lambda i: (i, 0, 0)),
            pl.BlockSpec((1, 64), lambda i: (0, 0)),
            pl.BlockSpec((1, 64), lambda i: (0, 0)),
            pl.BlockSpec((gb, 12, 64), lambda i: (i, 0, 0)),
            pl.BlockSpec((1, 64), lambda i: (0, 0)),
            pl.BlockSpec((1, 64), lambda i: (0, 0)),
            pl.BlockSpec((128, 5), lambda i: (0, 0)),
            pl.BlockSpec((128, 5), lambda i: (0, 0)),
            pl.BlockSpec((256, 5), lambda i: (0, 0)),
            pl.BlockSpec((1, 5), lambda i: (0, 0)),
        ],
        out_specs=[
            pl.BlockSpec((gb, 5), lambda i: (i, 0)),
            pl.BlockSpec((gb, 5), lambda i: (i, 0)),
            pl.BlockSpec((gb, 256), lambda i: (i, 0)),
            pl.BlockSpec((gb, 5), lambda i: (i, 0)),
        ],
        out_shape=[
            jax.ShapeDtypeStruct((BB, 5), jnp.float32),
            jax.ShapeDtypeStruct((BB, 5), jnp.float32),
            jax.ShapeDtypeStruct((BB, 256), jnp.float32),
            jax.ShapeDtypeStruct((BB, 5), jnp.float32),
        ],
    )(xsF3, smF, sqF, xsD3, smD, sqD, linFT, linDT, linT, lb2)


EP = EE // 16
NSL = EP // 128


def _fill_const(ref, n, val, dtype):
    def body(i, _):
        ref[pl.ds(i * 16, 16)] = jnp.full((16,), val, dtype)
        return 0
    lax.fori_loop(0, n // 16, body, 0)


def _deg_kernel(rows):
    mesh = plsc.VectorSubcoreMesh(core_axis_name="c", subcore_axis_name="s")

    @functools.partial(
        pl.kernel, mesh=mesh,
        compiler_params=pltpu.CompilerParams(needs_layout_passes=False),
        out_type=jax.ShapeDtypeStruct((2, NN), jnp.float32),
        scratch_types=[
            pltpu.VMEM((NSL, 128), jnp.int32),
            pltpu.VMEM((128,), jnp.float32),
            pltpu.VMEM((NN // 16,), jnp.float32),
            pltpu.VMEM_SHARED((NN,), jnp.float32),
            pltpu.SemaphoreType.DMA,
        ],
    )
    def k(r_hbm, d_hbm, idx2, ones_v, zrow, deg_sh, sem):
        c = lax.axis_index("c")
        s = lax.axis_index("s")
        _fill_const(ones_v, 128, 1.0, jnp.float32)
        _fill_const(zrow, NN // 16, 0.0, jnp.float32)
        pltpu.sync_copy(r_hbm.at[c, s], idx2)
        pltpu.sync_copy(zrow, deg_sh.at[pl.ds(s * (NN // 16), NN // 16)])
        plsc.subcore_barrier()
        for j in range(NSL):
            pltpu.async_copy(ones_v, deg_sh.at[idx2.at[j]], sem, add=True)
        for j in range(NSL):
            pltpu.make_async_copy(ones_v, deg_sh.at[idx2.at[j]], sem).wait()
        plsc.subcore_barrier()
        pltpu.sync_copy(deg_sh.at[pl.ds(s * (NN // 16), NN // 16)],
                        d_hbm.at[c, pl.ds(s * (NN // 16), NN // 16)])

    return k(rows)


EPW = 4608
NW = EE // EPW
CAPL = 4864


def _scatter_kernel(fo, fg, rr, y, row, col):
    npass = (NN // 32) // rr
    GS = 32 if fo >= 256 else 64
    mesh = plsc.VectorSubcoreMesh(core_axis_name="c", subcore_axis_name="s")

    @functools.partial(
        pl.kernel, mesh=mesh,
        compiler_params=pltpu.CompilerParams(needs_layout_passes=False),
        out_type=jax.ShapeDtypeStruct((NN * fo,), jnp.float32),
        scratch_types=[
            pltpu.VMEM((EPW,), jnp.int32),
            pltpu.VMEM((EPW,), jnp.int32),
            pltpu.VMEM((CAPL + 128,), jnp.int32),
            pltpu.VMEM((CAPL + 128,), jnp.int32),
            pltpu.VMEM((CAPL + 128,), jnp.int32),
            pltpu.VMEM((GS,), jnp.int32),
            pltpu.VMEM((GS,), jnp.int32),
            pltpu.VMEM((GS, fg), jnp.float32),
            pltpu.VMEM((GS, fg), jnp.float32),
            pltpu.VMEM(((rr + 32) * fo,), jnp.float32),
            pltpu.SemaphoreType.DMA,
            pltpu.SemaphoreType.DMA,
            pltpu.SemaphoreType.DMA,
            pltpu.SemaphoreType.DMA,
            pltpu.SemaphoreType.DMA,
        ],
    )
    def k(y_hbm, r_hbm, c_hbm, o_hbm,
          colbA, colbB, cpos, cloc, crv, srowA, srowB, gbufA, gbufB, acc,
          semA, semB, semR, semWA, semWB):
        c = lax.axis_index("c")
        s = lax.axis_index("s")
        t = s * 2 + c
        lanes = lax.broadcasted_iota(jnp.int32, (16,), 0)

        def load_idx(g, sr):
            for q in range(GS // 16):
                sr[pl.ds(q * 16, 16)] = crv[pl.ds(g * GS + q * 16, 16)]

        def addg(g, gbuf):
            for q in range(GS // 16):
                dv = cloc[pl.ds(g * GS + q * 16, 16)] * fo

                for l in range(16):
                    dst = dv[l]

                    def col_add(kq, _):
                        plsc.addupdate(
                            acc.at[pl.ds(dst + kq * 16, 16)],
                            gbuf[q * 16 + l, pl.ds(kq * 16, 16)])
                        return 0

                    lax.fori_loop(0, fo // 16, col_add, 0, unroll=4)

        def flush(cnt):
            for q in range(4):
                cpos[pl.ds(cnt + q * 16, 16)] = (
                    (q * 16 + lanes) * 2048 + rr + (q % 2) * 16 + lanes)

            def unpack(j, _):
                pk = cpos[pl.ds(j * 16, 16)]
                cpos[pl.ds(j * 16, 16)] = lax.shift_right_logical(pk, 11)
                cloc[pl.ds(j * 16, 16)] = lax.bitwise_and(pk, 2047)
                return 0

            lax.fori_loop(0, (cnt + 64 + 15) // 16, unpack, 0)
            nrg = (cnt + 127) // 128

            def rg(g, _):
                pltpu.async_copy(
                    r_hbm.at[cpos.at[pl.ds(g * 128, 128)]],
                    crv.at[pl.ds(g * 128, 128)], semR)
                return 0

            lax.fori_loop(0, nrg, rg, 0)

            def rgw(g, _):
                pltpu.make_async_copy(
                    r_hbm.at[cpos.at[pl.ds(g * 128, 128)]],
                    crv.at[pl.ds(g * 128, 128)], semR).wait()
                return 0

            lax.fori_loop(0, nrg, rgw, 0)
            nsl = (cnt + GS - 1) // GS
            load_idx(0, srowA)
            pltpu.async_copy(y_hbm.at[srowA], gbufA, semA)

            def sl(g, _):
                @pl.when(g % 2 == 0)
                def _():
                    pltpu.make_async_copy(y_hbm.at[srowA], gbufA, semA).wait()

                    @pl.when(g + 1 < nsl)
                    def _():
                        load_idx(g + 1, srowB)
                        pltpu.async_copy(y_hbm.at[srowB], gbufB, semB)
                    addg(g, gbufA)

                @pl.when(g % 2 == 1)
                def _():
                    pltpu.make_async_copy(y_hbm.at[srowB], gbufB, semB).wait()

                    @pl.when(g + 1 < nsl)
                    def _():
                        load_idx(g + 1, srowA)
                        pltpu.async_copy(y_hbm.at[srowA], gbufA, semA)
                    addg(g, gbufB)

                return 0

            lax.fori_loop(0, nsl, sl, 0)

        def scan_win(w, colb, cntv, base):
            def comp(i, cntv):
                cv = colb[pl.ds(i * 16, 16)]
                mv = cv - base
                m = (mv >= 0) & (mv < rr)
                pos = cntv + plsc.cumsum(m.astype(jnp.int32)) - 1
                plsc.store_scatter(
                    cpos, [pos],
                    (w * EPW + i * 16 + lanes) * 2048 + mv, mask=m)
                return cntv + plsc.all_reduce_population_count(m)

            cntv = lax.fori_loop(0, EPW // 16, comp, cntv, unroll=4)
            cs = cntv[0]
            full = cs >= CAPL - EPW - 64

            @pl.when(full)
            def _():
                flush(cs)

            return jnp.where(full, 0, cntv)

        def one_pass(p, _):
            base = t * (NN // 32) + p * rr
            _fill_const(acc, (rr + 32) * fo, 0.0, jnp.float32)
            pltpu.async_copy(c_hbm.at[pl.ds(0, EPW)], colbA, semWA)

            def wpair(wp, cntv):
                w = 2 * wp
                pltpu.make_async_copy(
                    c_hbm.at[pl.ds(0, EPW)], colbA, semWA).wait()
                pltpu.async_copy(
                    c_hbm.at[pl.ds((w + 1) * EPW, EPW)], colbB, semWB)
                cntv = scan_win(w, colbA, cntv, base)
                pltpu.make_async_copy(
                    c_hbm.at[pl.ds(0, EPW)], colbB, semWB).wait()

                @pl.when(wp + 1 < NW // 2)
                def _():
                    pltpu.async_copy(
                        c_hbm.at[pl.ds((w + 2) * EPW, EPW)], colbA, semWA)
                cntv = scan_win(w + 1, colbB, cntv, base)
                return cntv

            cntv = lax.fori_loop(0, NW // 2, wpair,
                                 jnp.zeros((16,), jnp.int32))
            cs = cntv[0]

            @pl.when(cs > 0)
            def _():
                flush(cs)

            pltpu.sync_copy(acc.at[pl.ds(0, rr * fo)],
                            o_hbm.at[pl.ds(base * fo, rr * fo)])
            return 0

        lax.fori_loop(0, npass, one_pass, 0)

    return k(y, row, col).reshape(NN, fo)


def _emb_table(Temb, Semb):
    e = (Temb[:, None, :] + Semb[None, :, :]).reshape(36, DD)
    return jnp.tile(e, (RB // 36, 1))


def kernel(xF, edge_index_F, batch_F, A_F, xD, edge_index_D, batch_D, A_D,
           TembF, SembF, TembD, SembD, WF1, bF1, WF2, bF2, WD1, bD1,
           WD2, bD2, lin1_W, lin1_b):
    f32 = jnp.float32
    rowF = edge_index_F[0]
    colF = edge_index_F[1]
    rowD = edge_index_D[0]
    colD = edge_index_D[1]

    rows = jnp.stack([rowF, rowD]).reshape(2, 16, EE // (16 * 128), 128)
    deg = _deg_kernel(rows)
    degF2 = deg[0][:, None]
    degD2 = deg[1][:, None]

    wF1 = jnp.concatenate([WF1[0], WF1[1]], axis=1).astype(jnp.bfloat16)
    wD1 = jnp.concatenate([WD1[0], WD1[1]], axis=1).astype(jnp.bfloat16)
    bF1p = jnp.concatenate([bF1, jnp.zeros((256,), f32)])[None, :]
    bD1p = jnp.concatenate([bD1, jnp.zeros((256,), f32)])[None, :]
    embF8 = _embmm(_emb_table(TembF, SembF), wF1, bF1p, 256)
    embD8 = _embmm(_emb_table(TembD, SembD), wD1, bD1p, 256)

    u0F, ypF = _mm1(xF, wF1, embF8, degF2, 256)
    u0D, ypD = _mm1(xD, wD1, embD8, degD2, 256)

    sF = _scatter_kernel(256, 256, 288, ypF, rowF, colF)
    sD = _scatter_kernel(256, 256, 288, ypD, rowD, colD)

    hF, smF, sqF = _stats(u0F, sF, degF2, 256)
    hD, smD, sqD = _stats(u0D, sD, degD2, 256)

    wF2 = jnp.concatenate([WF2[0], WF2[1]], axis=1).astype(jnp.bfloat16)
    wD2 = jnp.concatenate([WD2[0], WD2[1]], axis=1).astype(jnp.bfloat16)
    bF2p = jnp.concatenate([bF2, jnp.zeros((64,), f32)])[None, :]
    bD2p = jnp.concatenate([bD2, jnp.zeros((64,), f32)])[None, :]

    u0F2, ypF2 = _mm2(hF, smF, sqF, wF2, bF2p, degF2, 64)
    u0D2, ypD2 = _mm2(hD, smD, sqD, wD2, bD2p, degD2, 64)

    sF2 = _scatter_kernel(64, 128, 1152, ypF2, rowF, colF)
    sD2 = _scatter_kernel(64, 128, 1152, ypD2, rowD, colD)

    xsF, smF2, sqF2 = _stats2(u0F2, sF2, degF2, 64)
    xsD, smD2, sqD2 = _stats2(u0D2, sD2, degD2, 64)

    linFT = lin1_W[:, :128].T
    linDT = lin1_W[:, 128:].T
    linT = lin1_W.T
    lb2 = lin1_b[None, :]

    outputF, outputD, fusion, output = _heads(
        xsF.reshape(BB, 12, 64), smF2, sqF2,
        xsD.reshape(BB, 12, 64), smD2, sqD2,
        linFT, linDT, linT, lb2)
    return (outputF, outputD, fusion, output)
